# baseline (speedup 1.0000x reference)
import functools

import jax
import jax.numpy as jnp
from jax import lax
from jax.experimental import pallas as pl
from jax.experimental.pallas import tpu as pltpu
from jax.experimental.pallas import tpu_sc as plsc

_NC, _NS, _L = 2, 16, 16


def _matmul_body(x_ref, w_ref, out_ref):
    out_ref[...] = jax.lax.dot_general(
        x_ref[...], w_ref[...],
        dimension_numbers=(((1,), (0,)), ((), ())),
        preferred_element_type=jnp.float32,
    )


def _router_body(logits_hbm, scores_hbm, idx_hbm, wts_hbm,
                 slab, scores_v, idx_v, wts_v, *, TPW, E, K):
    wid = lax.axis_index("s") * _NC + lax.axis_index("c")
    base = wid * TPW
    pltpu.sync_copy(logits_hbm.at[pl.ds(base * E, TPW * E)], slab)
    lane = lax.broadcasted_iota(jnp.int32, (_L,), 0)
    lane_e = lane * E
    zero16 = jnp.zeros((_L,), jnp.float32)
    ninf = jnp.full((_L,), -jnp.inf, jnp.float32)
    izero = jnp.zeros((_L,), jnp.int32)

    def group(g, carry):
        gbase = g * (_L * E)
        vals = [ninf] * K
        idxs = [izero] * K
        gidx0 = gbase + lane_e
        for e in range(E):
            cv = plsc.load_gather(slab, [gidx0 + e])
            ci = jnp.full((_L,), e, jnp.int32)
            take = jnp.zeros((_L,), jnp.bool_)
            for j in range(K):
                take = jnp.logical_or(take, cv > vals[j])
                nv = jnp.where(take, cv, vals[j])
                sv = jnp.where(take, vals[j], cv)
                ni = jnp.where(take, ci, idxs[j])
                si = jnp.where(take, idxs[j], ci)
                vals[j] = nv
                idxs[j] = ni
                cv = sv
                ci = si
        mx = vals[0]
        exps = [jnp.exp(v - mx) for v in vals]
        denom = exps[0]
        for j in range(1, K):
            denom = denom + exps[j]
        inv = 1.0 / denom
        for i in range(E):
            scores_v[pl.ds(gbase + i * _L, _L)] = zero16
        row_flat = gbase + lane_e
        out_flat = g * (_L * K) + lane * K
        for j in range(K):
            w = exps[j] * inv
            plsc.store_scatter(scores_v, [row_flat + idxs[j]], w)
            plsc.store_scatter(idx_v, [out_flat + j], idxs[j])
            plsc.store_scatter(wts_v, [out_flat + j], w)
        return carry

    lax.fori_loop(0, TPW // _L, group, None)
    pltpu.sync_copy(scores_v, scores_hbm.at[pl.ds(base * E, TPW * E)])
    pltpu.sync_copy(idx_v, idx_hbm.at[pl.ds(base * K, TPW * K)])
    pltpu.sync_copy(wts_v, wts_hbm.at[pl.ds(base * K, TPW * K)])


def kernel(x, W):
    B, S, H = x.shape
    E = W.shape[0]
    K = 8
    N = B * S
    NCHUNK = 4
    NT = N // NCHUNK
    T = 1024
    while NT % T:
        T //= 2
    xr = x.reshape(N, H)
    wt = W.T
    NW = _NC * _NS
    TPW = NT // NW

    matmul = pl.pallas_call(
        _matmul_body,
        grid=(NT // T,),
        in_specs=[
            pl.BlockSpec((T, H), lambda i: (i, 0)),
            pl.BlockSpec((H, E), lambda i: (0, 0)),
        ],
        out_specs=pl.BlockSpec((T, E), lambda i: (i, 0)),
        out_shape=jax.ShapeDtypeStruct((NT, E), jnp.float32),
    )
    router = pl.kernel(
        functools.partial(_router_body, TPW=TPW, E=E, K=K),
        out_type=[
            jax.ShapeDtypeStruct((NT * E,), jnp.float32),
            jax.ShapeDtypeStruct((NT * K,), jnp.int32),
            jax.ShapeDtypeStruct((NT * K,), jnp.float32),
        ],
        mesh=plsc.VectorSubcoreMesh(core_axis_name="c", subcore_axis_name="s",
                                    num_cores=_NC, num_subcores=_NS),
        compiler_params=pltpu.CompilerParams(needs_layout_passes=False),
        scratch_types=[
            pltpu.VMEM((TPW * E,), jnp.float32),
            pltpu.VMEM((TPW * E,), jnp.float32),
            pltpu.VMEM((TPW * K,), jnp.int32),
            pltpu.VMEM((TPW * K,), jnp.float32),
        ],
    )

    parts = []
    for c in range(NCHUNK):
        logits_c = matmul(jax.lax.slice_in_dim(xr, c * NT, (c + 1) * NT, axis=0), wt)
        parts.append(router(logits_c.reshape(NT * E)))
    scores = jnp.concatenate([p[0] for p in parts])
    idx = jnp.concatenate([p[1] for p in parts])
    wts = jnp.concatenate([p[2] for p in parts])
    return (scores.reshape(B, S, E), idx.reshape(B, S, K), wts.reshape(B, S, K))


# speedup vs baseline: 1.0238x; 1.0238x over previous
import functools

import jax
import jax.numpy as jnp
from jax import lax
from jax.experimental import pallas as pl
from jax.experimental.pallas import tpu as pltpu
from jax.experimental.pallas import tpu_sc as plsc

_NC, _NS, _L = 2, 16, 16


def _matmul_body(x_ref, w_ref, out_ref):
    out_ref[...] = jax.lax.dot_general(
        x_ref[...], w_ref[...],
        dimension_numbers=(((1,), (0,)), ((), ())),
        preferred_element_type=jnp.float32,
    )


def _router_body(logits_hbm, scores_hbm, idx_hbm, wts_hbm,
                 slab, scores_v, idx_v, wts_v, *, TPW, E, K):
    wid = lax.axis_index("s") * _NC + lax.axis_index("c")
    base = wid * TPW
    pltpu.sync_copy(logits_hbm.at[pl.ds(base * E, TPW * E)], slab)
    lane = lax.broadcasted_iota(jnp.int32, (_L,), 0)
    lane_e = lane * E
    zero16 = jnp.zeros((_L,), jnp.float32)
    ninf = jnp.full((_L,), -jnp.inf, jnp.float32)
    izero = jnp.zeros((_L,), jnp.int32)

    def group(g, carry):
        gbase = g * (_L * E)
        vals = [ninf] * K
        idxs = [izero] * K
        gidx0 = gbase + lane_e
        for e in range(E):
            cv = plsc.load_gather(slab, [gidx0 + e])
            ci = jnp.full((_L,), e, jnp.int32)
            take = jnp.zeros((_L,), jnp.bool_)
            for j in range(K):
                take = jnp.logical_or(take, cv > vals[j])
                nv = jnp.where(take, cv, vals[j])
                sv = jnp.where(take, vals[j], cv)
                ni = jnp.where(take, ci, idxs[j])
                si = jnp.where(take, idxs[j], ci)
                vals[j] = nv
                idxs[j] = ni
                cv = sv
                ci = si
        mx = vals[0]
        exps = [jnp.exp(v - mx) for v in vals]
        denom = exps[0]
        for j in range(1, K):
            denom = denom + exps[j]
        inv = 1.0 / denom
        for i in range(E):
            scores_v[pl.ds(gbase + i * _L, _L)] = zero16
        row_flat = gbase + lane_e
        out_flat = g * (_L * K) + lane * K
        for j in range(K):
            w = exps[j] * inv
            plsc.store_scatter(scores_v, [row_flat + idxs[j]], w)
            plsc.store_scatter(idx_v, [out_flat + j], idxs[j])
            plsc.store_scatter(wts_v, [out_flat + j], w)
        return carry

    lax.fori_loop(0, TPW // _L, group, None)
    pltpu.sync_copy(scores_v, scores_hbm.at[pl.ds(base * E, TPW * E)])
    pltpu.sync_copy(idx_v, idx_hbm.at[pl.ds(base * K, TPW * K)])
    pltpu.sync_copy(wts_v, wts_hbm.at[pl.ds(base * K, TPW * K)])


def kernel(x, W):
    B, S, H = x.shape
    E = W.shape[0]
    K = 8
    N = B * S
    NCHUNK = 2
    NT = N // NCHUNK
    T = 1024
    while NT % T:
        T //= 2
    xr = x.reshape(N, H)
    wt = W.T
    NW = _NC * _NS
    TPW = NT // NW

    matmul = pl.pallas_call(
        _matmul_body,
        grid=(NT // T,),
        in_specs=[
            pl.BlockSpec((T, H), lambda i: (i, 0)),
            pl.BlockSpec((H, E), lambda i: (0, 0)),
        ],
        out_specs=pl.BlockSpec((T, E), lambda i: (i, 0)),
        out_shape=jax.ShapeDtypeStruct((NT, E), jnp.float32),
    )
    router = pl.kernel(
        functools.partial(_router_body, TPW=TPW, E=E, K=K),
        out_type=[
            jax.ShapeDtypeStruct((NT * E,), jnp.float32),
            jax.ShapeDtypeStruct((NT * K,), jnp.int32),
            jax.ShapeDtypeStruct((NT * K,), jnp.float32),
        ],
        mesh=plsc.VectorSubcoreMesh(core_axis_name="c", subcore_axis_name="s",
                                    num_cores=_NC, num_subcores=_NS),
        compiler_params=pltpu.CompilerParams(needs_layout_passes=False),
        scratch_types=[
            pltpu.VMEM((TPW * E,), jnp.float32),
            pltpu.VMEM((TPW * E,), jnp.float32),
            pltpu.VMEM((TPW * K,), jnp.int32),
            pltpu.VMEM((TPW * K,), jnp.float32),
        ],
    )

    logits_cs = [
        matmul(jax.lax.slice_in_dim(xr, c * NT, (c + 1) * NT, axis=0), wt)
        for c in range(NCHUNK)
    ]
    parts = [router(lc.reshape(NT * E)) for lc in logits_cs]
    scores = jnp.concatenate([p[0] for p in parts])
    idx = jnp.concatenate([p[1] for p in parts])
    wts = jnp.concatenate([p[2] for p in parts])
    return (scores.reshape(B, S, E), idx.reshape(B, S, K), wts.reshape(B, S, K))


# speedup vs baseline: 1.7677x; 1.7266x over previous
import functools

import jax
import jax.numpy as jnp
from jax import lax
from jax.experimental import pallas as pl
from jax.experimental.pallas import tpu as pltpu
from jax.experimental.pallas import tpu_sc as plsc

_NC, _NS, _L = 2, 16, 16


def _matmul_body(x_ref, w_ref, out_ref):
    out_ref[...] = jax.lax.dot_general(
        x_ref[...], w_ref[...],
        dimension_numbers=(((1,), (0,)), ((), ())),
        preferred_element_type=jnp.float32,
    )


def _router_body(logits_hbm, scores_hbm, idx_hbm, wts_hbm,
                 slab, scores_v, idx_v, wts_v, *, TPW, E, K):
    wid = lax.axis_index("s") * _NC + lax.axis_index("c")
    base = wid * TPW
    pltpu.sync_copy(logits_hbm.at[pl.ds(base * E, TPW * E)], slab)
    lane = lax.broadcasted_iota(jnp.int32, (_L,), 0)
    lane_e = lane * E
    zero16 = jnp.zeros((_L,), jnp.float32)
    ninf = jnp.full((_L,), -jnp.inf, jnp.float32)
    izero = jnp.zeros((_L,), jnp.int32)

    def group(g, carry):
        gbase = g * (_L * E)
        vals = [ninf] * K
        idxs = [izero] * K
        gidx0 = gbase + lane_e
        for e in range(E):
            cv = plsc.load_gather(slab, [gidx0 + e])
            ci = jnp.full((_L,), e, jnp.int32)
            take = jnp.zeros((_L,), jnp.bool_)
            for j in range(K):
                take = jnp.logical_or(take, cv > vals[j])
                nv = jnp.where(take, cv, vals[j])
                sv = jnp.where(take, vals[j], cv)
                ni = jnp.where(take, ci, idxs[j])
                si = jnp.where(take, idxs[j], ci)
                vals[j] = nv
                idxs[j] = ni
                cv = sv
                ci = si
        mx = vals[0]
        exps = [jnp.exp(v - mx) for v in vals]
        denom = exps[0]
        for j in range(1, K):
            denom = denom + exps[j]
        inv = 1.0 / denom
        for i in range(E):
            scores_v[pl.ds(gbase + i * _L, _L)] = zero16
        row_flat = gbase + lane_e
        out_flat = g * (_L * K) + lane * K
        for j in range(K):
            w = exps[j] * inv
            plsc.store_scatter(scores_v, [row_flat + idxs[j]], w)
            plsc.store_scatter(idx_v, [out_flat + j], idxs[j])
            plsc.store_scatter(wts_v, [out_flat + j], w)
        return carry

    lax.fori_loop(0, TPW // _L, group, None)
    pltpu.sync_copy(scores_v, scores_hbm.at[pl.ds(base * E, TPW * E)])
    pltpu.sync_copy(idx_v, idx_hbm.at[pl.ds(base * K, TPW * K)])
    pltpu.sync_copy(wts_v, wts_hbm.at[pl.ds(base * K, TPW * K)])


def kernel(x, W):
    B, S, H = x.shape
    E = W.shape[0]
    K = 8
    N = B * S
    T = 1024
    while N % T:
        T //= 2
    xr = x.reshape(N, H)
    wt = W.T
    logits = pl.pallas_call(
        _matmul_body,
        grid=(N // T,),
        in_specs=[
            pl.BlockSpec((T, H), lambda i: (i, 0)),
            pl.BlockSpec((H, E), lambda i: (0, 0)),
        ],
        out_specs=pl.BlockSpec((T, E), lambda i: (i, 0)),
        out_shape=jax.ShapeDtypeStruct((N, E), jnp.float32),
    )(xr, wt)

    NW = _NC * _NS
    TPW = N // NW
    router = pl.kernel(
        functools.partial(_router_body, TPW=TPW, E=E, K=K),
        out_type=[
            jax.ShapeDtypeStruct((N * E,), jnp.float32),
            jax.ShapeDtypeStruct((N * K,), jnp.int32),
            jax.ShapeDtypeStruct((N * K,), jnp.float32),
        ],
        mesh=plsc.VectorSubcoreMesh(core_axis_name="c", subcore_axis_name="s",
                                    num_cores=_NC, num_subcores=_NS),
        compiler_params=pltpu.CompilerParams(needs_layout_passes=False),
        scratch_types=[
            pltpu.VMEM((TPW * E,), jnp.float32),
            pltpu.VMEM((TPW * E,), jnp.float32),
            pltpu.VMEM((TPW * K,), jnp.int32),
            pltpu.VMEM((TPW * K,), jnp.float32),
        ],
    )
    scores, idx, wts = router(logits.reshape(N * E))
    return (scores.reshape(B, S, E), idx.reshape(B, S, K), wts.reshape(B, S, K))


# speedup vs baseline: 1.7695x; 1.0010x over previous
import functools

import jax
import jax.numpy as jnp
from jax import lax
from jax.experimental import pallas as pl
from jax.experimental.pallas import tpu as pltpu
from jax.experimental.pallas import tpu_sc as plsc

_NC, _NS, _L = 2, 16, 16


def _matmul_body(x_ref, w_ref, out_ref):
    out_ref[...] = jax.lax.dot_general(
        x_ref[...], w_ref[...],
        dimension_numbers=(((1,), (0,)), ((), ())),
        preferred_element_type=jnp.float32,
    )


def _router_body(logits_hbm, scores_hbm, idx_hbm, wts_hbm,
                 slab, scores_v, idx_v, wts_v, *, TPW, E, K):
    wid = lax.axis_index("s") * _NC + lax.axis_index("c")
    base = wid * TPW
    pltpu.sync_copy(logits_hbm.at[pl.ds(base * E, TPW * E)], slab)
    lane = lax.broadcasted_iota(jnp.int32, (_L,), 0)
    lane_e = lane * E
    zero16 = jnp.zeros((_L,), jnp.float32)
    ninf = jnp.full((_L,), -jnp.inf, jnp.float32)
    izero = jnp.zeros((_L,), jnp.int32)

    def group(g, carry):
        gbase = g * (_L * E)
        vals = [ninf] * K
        idxs = [izero] * K
        gidx0 = gbase + lane_e
        for e in range(E):
            cv = plsc.load_gather(slab, [gidx0 + e])
            ci = jnp.full((_L,), e, jnp.int32)
            take = cv > vals[0]
            for j in range(K):
                if j:
                    take = jnp.logical_or(take, cv > vals[j])
                nv = jnp.where(take, cv, vals[j])
                ni = jnp.where(take, ci, idxs[j])
                if j < K - 1:
                    sv = jnp.where(take, vals[j], cv)
                    si = jnp.where(take, idxs[j], ci)
                vals[j] = nv
                idxs[j] = ni
                if j < K - 1:
                    cv = sv
                    ci = si
        mx = vals[0]
        exps = [jnp.exp(v - mx) for v in vals]
        denom = exps[0]
        for j in range(1, K):
            denom = denom + exps[j]
        inv = 1.0 / denom
        for i in range(E):
            scores_v[pl.ds(gbase + i * _L, _L)] = zero16
        row_flat = gbase + lane_e
        out_flat = g * (_L * K) + lane * K
        for j in range(K):
            w = exps[j] * inv
            plsc.store_scatter(scores_v, [row_flat + idxs[j]], w)
            plsc.store_scatter(idx_v, [out_flat + j], idxs[j])
            plsc.store_scatter(wts_v, [out_flat + j], w)
        return carry

    lax.fori_loop(0, TPW // _L, group, None)
    pltpu.sync_copy(scores_v, scores_hbm.at[pl.ds(base * E, TPW * E)])
    pltpu.sync_copy(idx_v, idx_hbm.at[pl.ds(base * K, TPW * K)])
    pltpu.sync_copy(wts_v, wts_hbm.at[pl.ds(base * K, TPW * K)])


def kernel(x, W):
    B, S, H = x.shape
    E = W.shape[0]
    K = 8
    N = B * S
    T = 1024
    while N % T:
        T //= 2
    xr = x.reshape(N, H)
    wt = W.T
    logits = pl.pallas_call(
        _matmul_body,
        grid=(N // T,),
        in_specs=[
            pl.BlockSpec((T, H), lambda i: (i, 0)),
            pl.BlockSpec((H, E), lambda i: (0, 0)),
        ],
        out_specs=pl.BlockSpec((T, E), lambda i: (i, 0)),
        out_shape=jax.ShapeDtypeStruct((N, E), jnp.float32),
    )(xr, wt)

    NW = _NC * _NS
    TPW = N // NW
    router = pl.kernel(
        functools.partial(_router_body, TPW=TPW, E=E, K=K),
        out_type=[
            jax.ShapeDtypeStruct((N * E,), jnp.float32),
            jax.ShapeDtypeStruct((N * K,), jnp.int32),
            jax.ShapeDtypeStruct((N * K,), jnp.float32),
        ],
        mesh=plsc.VectorSubcoreMesh(core_axis_name="c", subcore_axis_name="s",
                                    num_cores=_NC, num_subcores=_NS),
        compiler_params=pltpu.CompilerParams(needs_layout_passes=False),
        scratch_types=[
            pltpu.VMEM((TPW * E,), jnp.float32),
            pltpu.VMEM((TPW * E,), jnp.float32),
            pltpu.VMEM((TPW * K,), jnp.int32),
            pltpu.VMEM((TPW * K,), jnp.float32),
        ],
    )
    scores, idx, wts = router(logits.reshape(N * E))
    return (scores.reshape(B, S, E), idx.reshape(B, S, K), wts.reshape(B, S, K))
